# baseline (speedup 1.0000x reference)
import jax
import jax.numpy as jnp
from jax.experimental import pallas as pl
from jax.experimental.pallas import tpu as pltpu

_B = 1_000_000
_NBOX = 6
_LBLK = 126976
_CORES = 2
_JSTEPS = 4


_CHUNK = 2048


def _iou_body(p_ref, t_ref, sum_ref, cnt_ref):
    i = pl.program_id(0)
    j = pl.program_id(1)

    lane = jax.lax.broadcasted_iota(jnp.int32, (1, _LBLK), 1)
    base = (i * _JSTEPS + j) * _LBLK
    in_bounds = (base + lane) < _B

    @pl.when(j == 0)
    def _():
        sum_ref[...] = jnp.zeros_like(sum_ref)
        cnt_ref[...] = jnp.zeros_like(cnt_ref)

    for k in range(_LBLK // _CHUNK):
        lo, hi = k * _CHUNK, (k + 1) * _CHUNK
        ib = in_bounds[:, lo:hi]
        sk = jnp.zeros((1, _CHUNK), jnp.float32)
        ck = jnp.zeros((1, _CHUNK), jnp.float32)
        for b in range(_NBOX):
            pcx = p_ref[b, 0:1, lo:hi]
            pcy = p_ref[b, 1:2, lo:hi]
            pw = p_ref[b, 2:3, lo:hi]
            ph = p_ref[b, 3:4, lo:hi]
            tcx = t_ref[b, 0:1, lo:hi]
            tcy = t_ref[b, 1:2, lo:hi]
            tw = t_ref[b, 2:3, lo:hi]
            th = t_ref[b, 3:4, lo:hi]

            px1 = pcx - 0.5 * pw
            px2 = pcx + 0.5 * pw
            py1 = pcy - 0.5 * ph
            py2 = pcy + 0.5 * ph
            tx1 = tcx - 0.5 * tw
            tx2 = tcx + 0.5 * tw
            ty1 = tcy - 0.5 * th
            ty2 = tcy + 0.5 * th

            dx = jnp.maximum(
                jnp.minimum(px2, tx2) - jnp.maximum(px1, tx1), 0.0
            )
            dy = jnp.maximum(
                jnp.minimum(py2, ty2) - jnp.maximum(py1, ty1), 0.0
            )
            inter = dx * dy
            area_p = jnp.abs(pw * ph)
            area_t = jnp.abs(tw * th)
            iou = inter / (area_p + area_t - inter + 1e-6)

            valid = (tcx != -1.0) & ib
            sk = sk + jnp.where(valid, iou, 0.0)
            ck = ck + jnp.where(valid, 1.0, 0.0)
        sum_ref[0, :, lo:hi] += sk
        cnt_ref[0, :, lo:hi] += ck


def kernel(pred, truth):
    p3 = jnp.transpose(pred, (1, 2, 0))
    t3 = jnp.transpose(truth, (1, 2, 0))

    in_spec = pl.BlockSpec(
        (_NBOX, 4, _LBLK), lambda i, j: (0, 0, i * _JSTEPS + j)
    )
    out_spec = pl.BlockSpec((1, 1, _LBLK), lambda i, j: (i, 0, 0))
    sums, cnts = pl.pallas_call(
        _iou_body,
        grid=(_CORES, _JSTEPS),
        in_specs=[in_spec, in_spec],
        out_specs=[out_spec, out_spec],
        out_shape=[
            jax.ShapeDtypeStruct((_CORES, 1, _LBLK), jnp.float32),
            jax.ShapeDtypeStruct((_CORES, 1, _LBLK), jnp.float32),
        ],
        compiler_params=pltpu.CompilerParams(
            dimension_semantics=("parallel", "arbitrary"),
        ),
    )(p3, t3)

    total_iou = jnp.sum(sums)
    n_valid = jnp.sum(cnts)
    mean_iou = jnp.where(
        n_valid > 0, total_iou / jnp.maximum(n_valid, 1.0), 0.0
    )
    return mean_iou.reshape(1, 1)


# speedup vs baseline: 1.0312x; 1.0312x over previous
import jax
import jax.numpy as jnp
from jax.experimental import pallas as pl
from jax.experimental.pallas import tpu as pltpu

_B = 1_000_000
_NBOX = 6
_LBLK = 65536
_CORES = 2
_JSTEPS = 8


_CHUNK = 2048


def _iou_body(p_ref, t_ref, sum_ref, cnt_ref):
    i = pl.program_id(0)
    j = pl.program_id(1)

    lane = jax.lax.broadcasted_iota(jnp.int32, (1, _LBLK), 1)
    base = (i * _JSTEPS + j) * _LBLK
    in_bounds = (base + lane) < _B

    @pl.when(j == 0)
    def _():
        sum_ref[...] = jnp.zeros_like(sum_ref)
        cnt_ref[...] = jnp.zeros_like(cnt_ref)

    for k in range(_LBLK // _CHUNK):
        lo, hi = k * _CHUNK, (k + 1) * _CHUNK
        ib = in_bounds[:, lo:hi]
        sk = jnp.zeros((1, _CHUNK), jnp.float32)
        ck = jnp.zeros((1, _CHUNK), jnp.float32)
        for b in range(_NBOX):
            pcx = p_ref[b, 0:1, lo:hi]
            pcy = p_ref[b, 1:2, lo:hi]
            pw = p_ref[b, 2:3, lo:hi]
            ph = p_ref[b, 3:4, lo:hi]
            tcx = t_ref[b, 0:1, lo:hi]
            tcy = t_ref[b, 1:2, lo:hi]
            tw = t_ref[b, 2:3, lo:hi]
            th = t_ref[b, 3:4, lo:hi]

            px1 = pcx - 0.5 * pw
            px2 = pcx + 0.5 * pw
            py1 = pcy - 0.5 * ph
            py2 = pcy + 0.5 * ph
            tx1 = tcx - 0.5 * tw
            tx2 = tcx + 0.5 * tw
            ty1 = tcy - 0.5 * th
            ty2 = tcy + 0.5 * th

            dx = jnp.maximum(
                jnp.minimum(px2, tx2) - jnp.maximum(px1, tx1), 0.0
            )
            dy = jnp.maximum(
                jnp.minimum(py2, ty2) - jnp.maximum(py1, ty1), 0.0
            )
            inter = dx * dy
            area_p = jnp.abs(pw * ph)
            area_t = jnp.abs(tw * th)
            iou = inter / (area_p + area_t - inter + 1e-6)

            valid = (tcx != -1.0) & ib
            sk = sk + jnp.where(valid, iou, 0.0)
            ck = ck + jnp.where(valid, 1.0, 0.0)
        sum_ref[0, :, lo:hi] += sk
        cnt_ref[0, :, lo:hi] += ck


def kernel(pred, truth):
    p3 = jnp.transpose(pred, (1, 2, 0))
    t3 = jnp.transpose(truth, (1, 2, 0))

    in_spec = pl.BlockSpec(
        (_NBOX, 4, _LBLK), lambda i, j: (0, 0, i * _JSTEPS + j)
    )
    out_spec = pl.BlockSpec((1, 1, _LBLK), lambda i, j: (i, 0, 0))
    sums, cnts = pl.pallas_call(
        _iou_body,
        grid=(_CORES, _JSTEPS),
        in_specs=[in_spec, in_spec],
        out_specs=[out_spec, out_spec],
        out_shape=[
            jax.ShapeDtypeStruct((_CORES, 1, _LBLK), jnp.float32),
            jax.ShapeDtypeStruct((_CORES, 1, _LBLK), jnp.float32),
        ],
        compiler_params=pltpu.CompilerParams(
            dimension_semantics=("parallel", "arbitrary"),
        ),
    )(p3, t3)

    total_iou = jnp.sum(sums)
    n_valid = jnp.sum(cnts)
    mean_iou = jnp.where(
        n_valid > 0, total_iou / jnp.maximum(n_valid, 1.0), 0.0
    )
    return mean_iou.reshape(1, 1)


# speedup vs baseline: 1.0623x; 1.0302x over previous
import jax
import jax.numpy as jnp
from jax.experimental import pallas as pl
from jax.experimental.pallas import tpu as pltpu

_B = 1_000_000
_NBOX = 6
_LBLK = 63488
_CORES = 2
_JSTEPS = 8


_CHUNK = 2048


def _iou_body(p_ref, t_ref, sum_ref, cnt_ref):
    i = pl.program_id(0)
    j = pl.program_id(1)

    lane = jax.lax.broadcasted_iota(jnp.int32, (1, _LBLK), 1)
    base = (i * _JSTEPS + j) * _LBLK
    in_bounds = (base + lane) < _B

    @pl.when(j == 0)
    def _():
        sum_ref[...] = jnp.zeros_like(sum_ref)
        cnt_ref[...] = jnp.zeros_like(cnt_ref)

    for k in range(_LBLK // _CHUNK):
        lo, hi = k * _CHUNK, (k + 1) * _CHUNK
        ib = in_bounds[:, lo:hi]
        sk = jnp.zeros((1, _CHUNK), jnp.float32)
        ck = jnp.zeros((1, _CHUNK), jnp.float32)
        for b in range(_NBOX):
            pcx = p_ref[b, 0:1, lo:hi]
            pcy = p_ref[b, 1:2, lo:hi]
            pw = p_ref[b, 2:3, lo:hi]
            ph = p_ref[b, 3:4, lo:hi]
            tcx = t_ref[b, 0:1, lo:hi]
            tcy = t_ref[b, 1:2, lo:hi]
            tw = t_ref[b, 2:3, lo:hi]
            th = t_ref[b, 3:4, lo:hi]

            px1 = pcx - 0.5 * pw
            px2 = pcx + 0.5 * pw
            py1 = pcy - 0.5 * ph
            py2 = pcy + 0.5 * ph
            tx1 = tcx - 0.5 * tw
            tx2 = tcx + 0.5 * tw
            ty1 = tcy - 0.5 * th
            ty2 = tcy + 0.5 * th

            dx = jnp.maximum(
                jnp.minimum(px2, tx2) - jnp.maximum(px1, tx1), 0.0
            )
            dy = jnp.maximum(
                jnp.minimum(py2, ty2) - jnp.maximum(py1, ty1), 0.0
            )
            inter = dx * dy
            area_p = jnp.abs(pw * ph)
            area_t = jnp.abs(tw * th)
            iou = inter / (area_p + area_t - inter + 1e-6)

            valid = (tcx != -1.0) & ib
            sk = sk + jnp.where(valid, iou, 0.0)
            ck = ck + jnp.where(valid, 1.0, 0.0)
        sum_ref[0, :, lo:hi] += sk
        cnt_ref[0, :, lo:hi] += ck


def kernel(pred, truth):
    p3 = jnp.transpose(pred, (1, 2, 0))
    t3 = jnp.transpose(truth, (1, 2, 0))

    in_spec = pl.BlockSpec(
        (_NBOX, 4, _LBLK), lambda i, j: (0, 0, i * _JSTEPS + j)
    )
    out_spec = pl.BlockSpec((1, 1, _LBLK), lambda i, j: (i, 0, 0))
    sums, cnts = pl.pallas_call(
        _iou_body,
        grid=(_CORES, _JSTEPS),
        in_specs=[in_spec, in_spec],
        out_specs=[out_spec, out_spec],
        out_shape=[
            jax.ShapeDtypeStruct((_CORES, 1, _LBLK), jnp.float32),
            jax.ShapeDtypeStruct((_CORES, 1, _LBLK), jnp.float32),
        ],
        compiler_params=pltpu.CompilerParams(
            dimension_semantics=("parallel", "arbitrary"),
        ),
    )(p3, t3)

    total_iou = jnp.sum(sums)
    n_valid = jnp.sum(cnts)
    mean_iou = jnp.where(
        n_valid > 0, total_iou / jnp.maximum(n_valid, 1.0), 0.0
    )
    return mean_iou.reshape(1, 1)


# speedup vs baseline: 1.0918x; 1.0277x over previous
import jax
import jax.numpy as jnp
from jax.experimental import pallas as pl
from jax.experimental.pallas import tpu as pltpu

_B = 1_000_000
_NBOX = 6
_LBLK = 63488
_CORES = 2
_JSTEPS = 8


_CHUNK = 2048


def _iou_body(p_ref, t_ref, sum_ref, cnt_ref):
    i = pl.program_id(0)
    j = pl.program_id(1)

    lane = jax.lax.broadcasted_iota(jnp.int32, (1, _LBLK), 1)
    base = (i * _JSTEPS + j) * _LBLK
    in_bounds = (base + lane) < _B

    @pl.when(j == 0)
    def _():
        sum_ref[...] = jnp.zeros_like(sum_ref)
        cnt_ref[...] = jnp.zeros_like(cnt_ref)

    sk = jnp.zeros((1, _CHUNK), jnp.float32)
    ck = jnp.zeros((1, _CHUNK), jnp.float32)
    for k in range(_LBLK // _CHUNK):
        lo, hi = k * _CHUNK, (k + 1) * _CHUNK
        ib = in_bounds[:, lo:hi]
        for b in range(_NBOX):
            pcx = p_ref[b, 0:1, lo:hi]
            pcy = p_ref[b, 1:2, lo:hi]
            pw = p_ref[b, 2:3, lo:hi]
            ph = p_ref[b, 3:4, lo:hi]
            tcx = t_ref[b, 0:1, lo:hi]
            tcy = t_ref[b, 1:2, lo:hi]
            tw = t_ref[b, 2:3, lo:hi]
            th = t_ref[b, 3:4, lo:hi]

            px1 = pcx - 0.5 * pw
            px2 = pcx + 0.5 * pw
            py1 = pcy - 0.5 * ph
            py2 = pcy + 0.5 * ph
            tx1 = tcx - 0.5 * tw
            tx2 = tcx + 0.5 * tw
            ty1 = tcy - 0.5 * th
            ty2 = tcy + 0.5 * th

            dx = jnp.maximum(
                jnp.minimum(px2, tx2) - jnp.maximum(px1, tx1), 0.0
            )
            dy = jnp.maximum(
                jnp.minimum(py2, ty2) - jnp.maximum(py1, ty1), 0.0
            )
            inter = dx * dy
            area_p = jnp.abs(pw * ph)
            area_t = jnp.abs(tw * th)
            iou = inter / (area_p + area_t - inter + 1e-6)

            valid = (tcx != -1.0) & ib
            sk = sk + jnp.where(valid, iou, 0.0)
            ck = ck + jnp.where(valid, 1.0, 0.0)
    sum_ref[0] += sk
    cnt_ref[0] += ck


def kernel(pred, truth):
    p3 = jnp.transpose(pred, (1, 2, 0))
    t3 = jnp.transpose(truth, (1, 2, 0))

    in_spec = pl.BlockSpec(
        (_NBOX, 4, _LBLK), lambda i, j: (0, 0, i * _JSTEPS + j)
    )
    out_spec = pl.BlockSpec((1, 1, _CHUNK), lambda i, j: (i, 0, 0))
    sums, cnts = pl.pallas_call(
        _iou_body,
        grid=(_CORES, _JSTEPS),
        in_specs=[in_spec, in_spec],
        out_specs=[out_spec, out_spec],
        out_shape=[
            jax.ShapeDtypeStruct((_CORES, 1, _CHUNK), jnp.float32),
            jax.ShapeDtypeStruct((_CORES, 1, _CHUNK), jnp.float32),
        ],
        compiler_params=pltpu.CompilerParams(
            dimension_semantics=("parallel", "arbitrary"),
        ),
    )(p3, t3)

    total_iou = jnp.sum(sums)
    n_valid = jnp.sum(cnts)
    mean_iou = jnp.where(
        n_valid > 0, total_iou / jnp.maximum(n_valid, 1.0), 0.0
    )
    return mean_iou.reshape(1, 1)
